# baseline (speedup 1.0000x reference)
import jax
import jax.numpy as jnp
from jax.experimental import pallas as pl
from jax.experimental.pallas import tpu as pltpu


C = 128
H1 = 64
W1 = 64
WH = W1 // 2
MH = H1 * WH
H2 = H1 // 2
M2 = H2 * WH
KP = 36
CE = 64
C3 = 96
HB = 2 * H1 + 2
NB = HB * WH
H2B = 2 * H2 + 1
H3B = 2 * H2 + 2


def _body(p_ref, swp, ssp, sbp,
          e1p, e1sp, e1bp, d1wp, d1sp, d1bp, p1p, p1sp, p1bp,
          e2p, e2sp, e2bp, d2wp, d2sp, d2bp, p2z, p2s, p2b,
          e3w, e3s, e3b, d3w, d3s, d3b, p3w, p3s, p3b,
          hw, hs, hb,
          o_ref,
          hP, hR, hp3):
    f32 = jnp.float32
    bf16 = jnp.bfloat16
    mask = jax.lax.broadcasted_iota(jnp.int32, (HB, WH, C), 2) < CE

    hP[0:1, :, :] = jnp.zeros((1, WH, C), f32)
    hP[H1 + 1:H1 + 3, :, :] = jnp.zeros((2, WH, C), f32)
    hP[2 * H1 + 3:2 * H1 + 4, :, :] = jnp.zeros((1, WH, C), f32)
    hR[0:1, :, :] = jnp.zeros((1, WH + 2, C), f32)
    hR[H1 + 1:H1 + 3, :, :] = jnp.zeros((2, WH + 2, C), f32)
    hR[2 * H1 + 3:2 * H1 + 4, :, :] = jnp.zeros((1, WH + 2, C), f32)
    hR[:, 0:1, :] = jnp.zeros((2 * H1 + 4, 1, C), f32)
    hR[:, WH + 1:WH + 2, :] = jnp.zeros((2 * H1 + 4, 1, C), f32)
    hp3[0:1, :, :] = jnp.zeros((1, H2 + 2, C3), f32)
    hp3[H2 + 1:H2 + 3, :, :] = jnp.zeros((2, H2 + 2, C3), f32)
    hp3[2 * H2 + 3:2 * H2 + 4, :, :] = jnp.zeros((1, H2 + 2, C3), f32)
    hp3[:, 0:1, :] = jnp.zeros((2 * H2 + 4, 1, C3), f32)
    hp3[:, H2 + 1:H2 + 2, :] = jnp.zeros((2 * H2 + 4, 1, C3), f32)

    s = jnp.dot(p_ref[...], swp[...], preferred_element_type=f32)
    spb = jnp.clip(s * ssp[...] + sbp[...], 0.0, 6.0).astype(bf16)

    def packed_dw_input(P, scale, bias):
        P = jnp.clip(P * scale + bias, 0.0, 6.0)
        R = jnp.concatenate([P[:, CE:C], P[:, 0:CE]], axis=1)
        P3 = P.reshape(HB, WH, C)
        R3 = R.reshape(HB, WH, C)
        hP[1:H1 + 1, 0:WH, :] = P3[0:H1]
        hP[H1 + 3:2 * H1 + 3, 0:WH, :] = P3[H1 + 2:HB]
        hR[1:H1 + 1, 1:WH + 1, :] = R3[0:H1]
        hR[H1 + 3:2 * H1 + 3, 1:WH + 1, :] = R3[H1 + 2:HB]

    packed_dw_input(jnp.dot(spb, e1p[...], preferred_element_type=f32),
                    e1sp[...], e1bp[...])
    dww = d1wp[...]
    acc = jnp.zeros((HB, WH, C), f32)
    for dy in range(3):
        rP = hP[dy:dy + HB, :, :]
        rR = hR[dy:dy + HB, :, :]
        A0 = jnp.where(mask, rR[:, 0:WH, :], rR[:, 1:WH + 1, :])
        A2 = jnp.where(mask, rR[:, 1:WH + 1, :], rR[:, 2:WH + 2, :])
        acc += (A0 * dww[3 * dy + 0] + rP * dww[3 * dy + 1]
                + A2 * dww[3 * dy + 2])
    acc = jnp.clip(acc * d1sp[...] + d1bp[...], 0.0, 6.0)
    OPb = acc.astype(bf16).reshape(NB, C)

    y = jnp.dot(OPb, p1p[...], preferred_element_type=f32)
    yp = ((y * p1sp[...] + p1bp[...]) + spb.astype(f32)).astype(bf16)

    packed_dw_input(jnp.dot(yp, e2p[...], preferred_element_type=f32),
                    e2sp[...], e2bp[...])
    pV = hP[...].reshape(H1 + 2, 2, WH, C)
    rV = hR[...].reshape(H1 + 2, 2, WH + 2, C)
    rowsP = (pV[:, 0][0:H2B], pV[:, 1][0:H2B], pV[:, 0][1:H2B + 1])
    rowsR = (rV[:, 0][0:H2B], rV[:, 1][0:H2B], rV[:, 0][1:H2B + 1])
    dww2 = d2wp[...]
    acc2 = jnp.zeros((H2B, WH, C), f32)
    for dy in range(3):
        rP, rR = rowsP[dy], rowsR[dy]
        acc2 += (rR[:, 0:WH, :] * dww2[3 * dy + 0]
                 + rP[:, 0:WH, :] * dww2[3 * dy + 1]
                 + rR[:, 1:WH + 1, :] * dww2[3 * dy + 2])
    acc2 = jnp.clip(acc2 * d2sp[...] + d2bp[...], 0.0, 6.0)
    d2b = acc2.astype(bf16).reshape(H2B * WH, C)

    b2 = jnp.dot(d2b, p2z[...], preferred_element_type=f32)
    b2v = (b2 * p2s[...] + p2b[...]).astype(bf16)

    h3 = jnp.dot(b2v, e3w[...], preferred_element_type=f32)
    h3 = jnp.clip(h3 * e3s[...] + e3b[...], 0.0, 6.0)
    hp3[1:H2 + 1, 1:H2 + 1, :] = h3[0:M2].reshape(H2, H2, C3)
    hp3[H2 + 3:2 * H2 + 3, 1:H2 + 1, :] = \
        h3[M2 + WH:2 * M2 + WH].reshape(H2, H2, C3)

    dww3 = d3w[...]
    acc3 = jnp.zeros((H3B, H2, C3), f32)
    for dy in range(3):
        row = hp3[dy:dy + H3B, :, :]
        for dx in range(3):
            acc3 += row[:, dx:dx + H2, :] * dww3[3 * dy + dx]
    acc3 = jnp.clip(acc3 * d3s[...] + d3b[...], 0.0, 6.0)
    d3v = acc3.astype(bf16).reshape(H3B * WH, C3)

    b2a = jnp.concatenate(
        [b2v[0:M2].astype(f32), jnp.zeros((2 * WH, 24), f32),
         b2v[M2 + WH:2 * M2 + WH].astype(f32)], axis=0)
    b3 = jnp.dot(d3v, p3w[...], preferred_element_type=f32)
    b3 = b3 * p3s[...] + p3b[...] + b2a

    hact = jnp.dot(b3.astype(bf16), hw[...], preferred_element_type=f32)
    hact = jnp.clip(hact * hs[...] + hb[...], 0.0, 6.0)
    o_ref[0:1, :, :] = (jnp.sum(hact[0:M2], axis=0, keepdims=True)
                        * (1.0 / M2)).astype(o_ref.dtype).reshape(1, 1, C)
    o_ref[1:2, :, :] = (jnp.sum(hact[M2 + 2 * WH:2 * M2 + 2 * WH], axis=0,
                                keepdims=True)
                        * (1.0 / M2)).astype(o_ref.dtype).reshape(1, 1, C)


def _build_patches(images):
    n = images.shape[0]
    x = jnp.transpose(images, (0, 2, 3, 1))
    xp = jnp.pad(x, ((0, 0), (1, 1), (1, 3), (0, 0)))
    wins = []
    for dy in range(3):
        g = xp[:, dy:dy + 2 * H1:2, :, :].reshape(n, H1, 66, 6)
        wins.append(jnp.concatenate([g[:, :, 0:W1, :], g[:, :, 1:W1 + 1, :]],
                                    axis=3))
    patches = jnp.concatenate(wins, axis=3)
    patches = patches.reshape(n // 2, 2, MH, 2 * KP)
    patches = jnp.pad(patches, ((0, 0), (0, 0), (0, 2 * WH), (0, 0)))
    return patches.reshape(n // 2, 2 * (MH + 2 * WH), 2 * KP)[:, :NB, :]


def _pack2(v):
    return jnp.concatenate([v[:, :CE], v[:, :CE]], axis=1)


def _pairdiag(w):
    z = jnp.zeros(w.shape, w.dtype)
    return jnp.concatenate(
        [jnp.concatenate([w, z], axis=1), jnp.concatenate([z, w], axis=1)],
        axis=0)


def kernel(images, stem_w, stem_s, stem_b,
           b1_exp_w, b1_exp_s, b1_exp_b, b1_dw_w, b1_dw_s, b1_dw_b,
           b1_proj_w, b1_proj_s, b1_proj_b,
           b2_exp_w, b2_exp_s, b2_exp_b, b2_dw_w, b2_dw_s, b2_dw_b,
           b2_proj_w, b2_proj_s, b2_proj_b,
           b3_exp_w, b3_exp_s, b3_exp_b, b3_dw_w, b3_dw_s, b3_dw_b,
           b3_proj_w, b3_proj_s, b3_proj_b,
           head_w, head_s, head_b):
    n = images.shape[0]
    pcat = _build_patches(images)

    sw36 = jnp.pad(stem_w[:27, :16].reshape(3, 3, 3, 16),
                   ((0, 0), (0, 1), (0, 0), (0, 0))).reshape(KP, 16)
    swp = _pairdiag(sw36)
    pair = lambda v: jnp.concatenate([v[:, :16], v[:, :16]], axis=1)
    ssp, sbp = pair(stem_s), pair(stem_b)
    e1p = _pairdiag(b1_exp_w[:16, :CE])
    e1sp, e1bp = _pack2(b1_exp_s), _pack2(b1_exp_b)
    d1wp = _pack2(b1_dw_w)
    d1sp, d1bp = _pack2(b1_dw_s), _pack2(b1_dw_b)
    p1p = _pairdiag(b1_proj_w[:CE, :16])
    p1sp, p1bp = pair(b1_proj_s), pair(b1_proj_b)
    e2p = _pairdiag(b2_exp_w[:16, :CE])
    e2sp, e2bp = _pack2(b2_exp_s), _pack2(b2_exp_b)
    d2wp = _pack2(b2_dw_w)
    d2sp, d2bp = _pack2(b2_dw_s), _pack2(b2_dw_b)
    p2z = jnp.concatenate(
        [b2_proj_w[:CE, :24], jnp.zeros((CE, 24), jnp.bfloat16)], axis=0)
    p2s, p2b = b2_proj_s[:, :24], b2_proj_b[:, :24]
    e3w = b3_exp_w[:24, :C3]
    e3s, e3b = b3_exp_s[:, :C3], b3_exp_b[:, :C3]
    d3w = b3_dw_w[:, :C3]
    d3s, d3b = b3_dw_s[:, :C3], b3_dw_b[:, :C3]
    p3w = b3_proj_w[:C3, :24]
    p3s, p3b = b3_proj_s[:, :24], b3_proj_b[:, :24]
    hw = head_w[:24, :]

    full = lambda i: (0, 0)
    ws = lambda r, c: pl.BlockSpec((r, c), full)

    out = pl.pallas_call(
        _body,
        grid=(n // 2,),
        in_specs=[pl.BlockSpec((None, NB, 2 * KP), lambda i: (i, 0, 0)),
                  ws(2 * KP, 32), ws(1, 32), ws(1, 32),
                  ws(32, C), ws(1, C), ws(1, C),
                  ws(9, C), ws(1, C), ws(1, C),
                  ws(C, 32), ws(1, 32), ws(1, 32),
                  ws(32, C), ws(1, C), ws(1, C),
                  ws(9, C), ws(1, C), ws(1, C),
                  ws(C, 24), ws(1, 24), ws(1, 24),
                  ws(24, C3), ws(1, C3), ws(1, C3),
                  ws(9, C3), ws(1, C3), ws(1, C3),
                  ws(C3, 24), ws(1, 24), ws(1, 24),
                  ws(24, C), ws(1, C), ws(1, C)],
        out_specs=pl.BlockSpec((2, 1, C), lambda i: (i, 0, 0)),
        out_shape=jax.ShapeDtypeStruct((n, 1, C), jnp.bfloat16),
        scratch_shapes=[
            pltpu.VMEM((2 * H1 + 4, WH, C), jnp.float32),
            pltpu.VMEM((2 * H1 + 4, WH + 2, C), jnp.float32),
            pltpu.VMEM((2 * H2 + 4, H2 + 2, C3), jnp.float32),
        ],
        compiler_params=pltpu.CompilerParams(
            dimension_semantics=("parallel",)),
    )(pcat, swp, ssp, sbp,
      e1p, e1sp, e1bp, d1wp, d1sp, d1bp, p1p, p1sp, p1bp,
      e2p, e2sp, e2bp, d2wp, d2sp, d2bp, p2z, p2s, p2b,
      e3w, e3s, e3b, d3w, d3s, d3b, p3w, p3s, p3b,
      hw, head_s, head_b)
    return out.astype(images.dtype)


# speedup vs baseline: 6.1963x; 6.1963x over previous
import jax
import jax.numpy as jnp
from jax.experimental import pallas as pl
from jax.experimental.pallas import tpu as pltpu


C = 128
H1 = 64
W1 = 64
WH = W1 // 2
M1 = H1 * W1
MH = M1 // 2
H2 = H1 // 2
M2 = H2 * WH
KP = 36
CE = 64
C3 = 96


def _body(p_ref, swp, ssp, sbp,
          e1p, e1sp, e1bp, d1wp, d1sp, d1bp, p1p, p1sp, p1bp,
          e2p, e2sp, e2bp, d2wp, d2sp, d2bp, p2z, p2s, p2b,
          e3w, e3s, e3b, d3w, d3s, d3b, p3w, p3s, p3b,
          hw, hs, hb,
          o_ref,
          hP, hR, hp3):
    f32 = jnp.float32
    bf16 = jnp.bfloat16
    mask = jax.lax.broadcasted_iota(jnp.int32, (H1, WH, C), 2) < CE

    hP[0:1, :, :] = jnp.zeros((1, WH, C), f32)
    hP[H1 + 1:H1 + 2, :, :] = jnp.zeros((1, WH, C), f32)
    hR[0:1, :, :] = jnp.zeros((1, WH + 2, C), f32)
    hR[H1 + 1:H1 + 2, :, :] = jnp.zeros((1, WH + 2, C), f32)
    hR[:, 0:1, :] = jnp.zeros((H1 + 2, 1, C), f32)
    hR[:, WH + 1:WH + 2, :] = jnp.zeros((H1 + 2, 1, C), f32)
    hp3[0:1, :, :] = jnp.zeros((1, H2 + 2, C3), f32)
    hp3[H2 + 1:H2 + 2, :, :] = jnp.zeros((1, H2 + 2, C3), f32)
    hp3[:, 0:1, :] = jnp.zeros((H2 + 2, 1, C3), f32)
    hp3[:, H2 + 1:H2 + 2, :] = jnp.zeros((H2 + 2, 1, C3), f32)

    s = jnp.dot(p_ref[...], swp[...], preferred_element_type=f32)
    spb = jnp.clip(s * ssp[...] + sbp[...], 0.0, 6.0).astype(bf16)

    def packed_dw_input(P, scale, bias):
        P = jnp.clip(P * scale + bias, 0.0, 6.0)
        R = jnp.concatenate([P[:, CE:C], P[:, 0:CE]], axis=1)
        hP[1:H1 + 1, 0:WH, :] = P.reshape(H1, WH, C)
        hR[1:H1 + 1, 1:WH + 1, :] = R.reshape(H1, WH, C)

    packed_dw_input(jnp.dot(spb, e1p[...], preferred_element_type=f32),
                    e1sp[...], e1bp[...])
    dww = d1wp[...]
    acc = jnp.zeros((H1, WH, C), f32)
    for dy in range(3):
        rP = hP[dy:dy + H1, :, :]
        rR = hR[dy:dy + H1, :, :]
        A0 = jnp.where(mask, rR[:, 0:WH, :], rR[:, 1:WH + 1, :])
        A2 = jnp.where(mask, rR[:, 1:WH + 1, :], rR[:, 2:WH + 2, :])
        acc += (A0 * dww[3 * dy + 0] + rP * dww[3 * dy + 1]
                + A2 * dww[3 * dy + 2])
    acc = jnp.clip(acc * d1sp[...] + d1bp[...], 0.0, 6.0)
    OPb = acc.astype(bf16).reshape(MH, C)

    y = jnp.dot(OPb, p1p[...], preferred_element_type=f32)
    yp = ((y * p1sp[...] + p1bp[...]) + spb.astype(f32)).astype(bf16)

    packed_dw_input(jnp.dot(yp, e2p[...], preferred_element_type=f32),
                    e2sp[...], e2bp[...])
    pV = hP[...].reshape((H1 + 2) // 2, 2, WH, C)
    rV = hR[...].reshape((H1 + 2) // 2, 2, WH + 2, C)
    rowsP = (pV[:, 0][0:H2], pV[:, 1][0:H2], pV[:, 0][1:H2 + 1])
    rowsR = (rV[:, 0][0:H2], rV[:, 1][0:H2], rV[:, 0][1:H2 + 1])
    dww2 = d2wp[...]
    acc2 = jnp.zeros((H2, WH, C), f32)
    for dy in range(3):
        rP, rR = rowsP[dy], rowsR[dy]
        acc2 += (rR[:, 0:WH, :] * dww2[3 * dy + 0]
                 + rP[:, 0:WH, :] * dww2[3 * dy + 1]
                 + rR[:, 1:WH + 1, :] * dww2[3 * dy + 2])
    acc2 = jnp.clip(acc2 * d2sp[...] + d2bp[...], 0.0, 6.0)
    d2b = acc2.astype(bf16).reshape(M2, C)

    b2 = jnp.dot(d2b, p2z[...], preferred_element_type=f32)
    b2v = (b2 * p2s[...] + p2b[...]).astype(bf16)

    h3 = jnp.dot(b2v, e3w[...], preferred_element_type=f32)
    h3 = jnp.clip(h3 * e3s[...] + e3b[...], 0.0, 6.0)
    hp3[1:H2 + 1, 1:H2 + 1, :] = h3.reshape(H2, H2, C3)

    dww3 = d3w[...]
    acc3 = jnp.zeros((H2, H2, C3), f32)
    for dy in range(3):
        row = hp3[dy:dy + H2, :, :]
        for dx in range(3):
            acc3 += row[:, dx:dx + H2, :] * dww3[3 * dy + dx]
    acc3 = jnp.clip(acc3 * d3s[...] + d3b[...], 0.0, 6.0)
    d3v = acc3.astype(bf16).reshape(M2, C3)

    b3 = jnp.dot(d3v, p3w[...], preferred_element_type=f32)
    b3 = b3 * p3s[...] + p3b[...] + b2v.astype(f32)

    hact = jnp.dot(b3.astype(bf16), hw[...], preferred_element_type=f32)
    hact = jnp.clip(hact * hs[...] + hb[...], 0.0, 6.0)
    o_ref[...] = (jnp.sum(hact, axis=0, keepdims=True)
                  * (1.0 / M2)).astype(o_ref.dtype)


def _build_patches(images):
    n = images.shape[0]
    x = jnp.transpose(images, (0, 2, 3, 1))
    xp = jnp.pad(x, ((0, 0), (1, 1), (1, 3), (0, 0)))
    wins = []
    for dy in range(3):
        g = xp[:, dy:dy + 2 * H1:2, :, :].reshape(n, H1, 66, 6)
        wins.append(jnp.concatenate([g[:, :, 0:W1, :], g[:, :, 1:W1 + 1, :]],
                                    axis=3))
    patches = jnp.concatenate(wins, axis=3)
    return patches.reshape(n, MH, 2 * KP)


def _pack2(v):
    return jnp.concatenate([v[:, :CE], v[:, :CE]], axis=1)


def _pairdiag(w, zeros_like_shape=None):
    z = jnp.zeros(w.shape, w.dtype)
    return jnp.concatenate(
        [jnp.concatenate([w, z], axis=1), jnp.concatenate([z, w], axis=1)],
        axis=0)


def kernel(images, stem_w, stem_s, stem_b,
           b1_exp_w, b1_exp_s, b1_exp_b, b1_dw_w, b1_dw_s, b1_dw_b,
           b1_proj_w, b1_proj_s, b1_proj_b,
           b2_exp_w, b2_exp_s, b2_exp_b, b2_dw_w, b2_dw_s, b2_dw_b,
           b2_proj_w, b2_proj_s, b2_proj_b,
           b3_exp_w, b3_exp_s, b3_exp_b, b3_dw_w, b3_dw_s, b3_dw_b,
           b3_proj_w, b3_proj_s, b3_proj_b,
           head_w, head_s, head_b):
    n = images.shape[0]
    pcat = _build_patches(images)

    sw36 = jnp.pad(stem_w[:27, :16].reshape(3, 3, 3, 16),
                   ((0, 0), (0, 1), (0, 0), (0, 0))).reshape(KP, 16)
    swp = _pairdiag(sw36)
    pair = lambda v: jnp.concatenate([v[:, :16], v[:, :16]], axis=1)
    ssp, sbp = pair(stem_s), pair(stem_b)
    e1p = _pairdiag(b1_exp_w[:16, :CE])
    e1sp, e1bp = _pack2(b1_exp_s), _pack2(b1_exp_b)
    d1wp = _pack2(b1_dw_w)
    d1sp, d1bp = _pack2(b1_dw_s), _pack2(b1_dw_b)
    p1p = _pairdiag(b1_proj_w[:CE, :16])
    p1sp, p1bp = pair(b1_proj_s), pair(b1_proj_b)
    e2p = _pairdiag(b2_exp_w[:16, :CE])
    e2sp, e2bp = _pack2(b2_exp_s), _pack2(b2_exp_b)
    d2wp = _pack2(b2_dw_w)
    d2sp, d2bp = _pack2(b2_dw_s), _pack2(b2_dw_b)
    p2z = jnp.concatenate(
        [b2_proj_w[:CE, :24], jnp.zeros((CE, 24), jnp.bfloat16)], axis=0)
    p2s, p2b = b2_proj_s[:, :24], b2_proj_b[:, :24]
    e3w = b3_exp_w[:24, :C3]
    e3s, e3b = b3_exp_s[:, :C3], b3_exp_b[:, :C3]
    d3w = b3_dw_w[:, :C3]
    d3s, d3b = b3_dw_s[:, :C3], b3_dw_b[:, :C3]
    p3w = b3_proj_w[:C3, :24]
    p3s, p3b = b3_proj_s[:, :24], b3_proj_b[:, :24]
    hw = head_w[:24, :]

    full = lambda i: (0, 0)
    ws = lambda r, c: pl.BlockSpec((r, c), full)

    out = pl.pallas_call(
        _body,
        grid=(n,),
        in_specs=[pl.BlockSpec((None, MH, 2 * KP), lambda i: (i, 0, 0)),
                  ws(2 * KP, 32), ws(1, 32), ws(1, 32),
                  ws(32, C), ws(1, C), ws(1, C),
                  ws(9, C), ws(1, C), ws(1, C),
                  ws(C, 32), ws(1, 32), ws(1, 32),
                  ws(32, C), ws(1, C), ws(1, C),
                  ws(9, C), ws(1, C), ws(1, C),
                  ws(C, 24), ws(1, 24), ws(1, 24),
                  ws(24, C3), ws(1, C3), ws(1, C3),
                  ws(9, C3), ws(1, C3), ws(1, C3),
                  ws(C3, 24), ws(1, 24), ws(1, 24),
                  ws(24, C), ws(1, C), ws(1, C)],
        out_specs=pl.BlockSpec((None, 1, C), lambda i: (i, 0, 0)),
        out_shape=jax.ShapeDtypeStruct((n, 1, C), jnp.bfloat16),
        scratch_shapes=[
            pltpu.VMEM((H1 + 2, WH, C), jnp.float32),
            pltpu.VMEM((H1 + 2, WH + 2, C), jnp.float32),
            pltpu.VMEM((H2 + 2, H2 + 2, C3), jnp.float32),
        ],
        compiler_params=pltpu.CompilerParams(
            dimension_semantics=("parallel",)),
    )(pcat, swp, ssp, sbp,
      e1p, e1sp, e1bp, d1wp, d1sp, d1bp, p1p, p1sp, p1bp,
      e2p, e2sp, e2bp, d2wp, d2sp, d2bp, p2z, p2s, p2b,
      e3w, e3s, e3b, d3w, d3s, d3b, p3w, p3s, p3b,
      hw, head_s, head_b)
    return out.astype(images.dtype)


# speedup vs baseline: 6.4436x; 1.0399x over previous
import jax
import jax.numpy as jnp
from jax.experimental import pallas as pl
from jax.experimental.pallas import tpu as pltpu


C = 128
H1 = 64
W1 = 64
WH = W1 // 2
M1 = H1 * W1
MH = M1 // 2
H2 = H1 // 2
M2 = H2 * WH
KP = 36
CE = 64
C3 = 96


def _body(p_ref, swp, ssp, sbp,
          e1p, e1sp, e1bp, d1wp, d1sp, d1bp, p1p, p1sp, p1bp,
          e2p, e2sp, e2bp, d2wp, d2sp, d2bp, p2z, p2s, p2b,
          e3w, e3s, e3b, d3w, d3s, d3b, p3w, p3s, p3b,
          hw, hs, hb,
          o_ref,
          hP, hR, hp3):
    f32 = jnp.float32
    bf16 = jnp.bfloat16
    mask = jax.lax.broadcasted_iota(jnp.int32, (2, H1, WH, C), 3) < CE

    hP[:, 0:1, :, :] = jnp.zeros((2, 1, WH, C), f32)
    hP[:, H1 + 1:H1 + 2, :, :] = jnp.zeros((2, 1, WH, C), f32)
    hR[:, 0:1, :, :] = jnp.zeros((2, 1, WH + 2, C), f32)
    hR[:, H1 + 1:H1 + 2, :, :] = jnp.zeros((2, 1, WH + 2, C), f32)
    hR[:, :, 0:1, :] = jnp.zeros((2, H1 + 2, 1, C), f32)
    hR[:, :, WH + 1:WH + 2, :] = jnp.zeros((2, H1 + 2, 1, C), f32)
    hp3[:, 0:1, :, :] = jnp.zeros((2, 1, H2 + 2, C3), f32)
    hp3[:, H2 + 1:H2 + 2, :, :] = jnp.zeros((2, 1, H2 + 2, C3), f32)
    hp3[:, :, 0:1, :] = jnp.zeros((2, H2 + 2, 1, C3), f32)
    hp3[:, :, H2 + 1:H2 + 2, :] = jnp.zeros((2, H2 + 2, 1, C3), f32)

    s = jnp.dot(p_ref[...].reshape(2 * MH, 2 * KP), swp[...],
                preferred_element_type=f32)
    spb = jnp.clip(s * ssp[...] + sbp[...], 0.0, 6.0).astype(bf16)

    def packed_dw_input(P, scale, bias):
        P = jnp.clip(P * scale + bias, 0.0, 6.0)
        R = jnp.concatenate([P[:, CE:C], P[:, 0:CE]], axis=1)
        hP[:, 1:H1 + 1, 0:WH, :] = P.reshape(2, H1, WH, C)
        hR[:, 1:H1 + 1, 1:WH + 1, :] = R.reshape(2, H1, WH, C)

    packed_dw_input(jnp.dot(spb, e1p[...], preferred_element_type=f32),
                    e1sp[...], e1bp[...])
    dww = d1wp[...]
    acc = jnp.zeros((2, H1, WH, C), f32)
    for dy in range(3):
        rP = hP[:, dy:dy + H1, :, :]
        rR = hR[:, dy:dy + H1, :, :]
        A0 = jnp.where(mask, rR[:, :, 0:WH, :], rR[:, :, 1:WH + 1, :])
        A2 = jnp.where(mask, rR[:, :, 1:WH + 1, :], rR[:, :, 2:WH + 2, :])
        acc += (A0 * dww[3 * dy + 0] + rP * dww[3 * dy + 1]
                + A2 * dww[3 * dy + 2])
    acc = jnp.clip(acc * d1sp[...] + d1bp[...], 0.0, 6.0)
    OPb = acc.astype(bf16).reshape(2 * MH, C)

    y = jnp.dot(OPb, p1p[...], preferred_element_type=f32)
    yp = ((y * p1sp[...] + p1bp[...]) + spb.astype(f32)).astype(bf16)

    packed_dw_input(jnp.dot(yp, e2p[...], preferred_element_type=f32),
                    e2sp[...], e2bp[...])
    pV = hP[...].reshape(2, (H1 + 2) // 2, 2, WH, C)
    rV = hR[...].reshape(2, (H1 + 2) // 2, 2, WH + 2, C)
    rowsP = (pV[:, :, 0][:, 0:H2], pV[:, :, 1][:, 0:H2],
             pV[:, :, 0][:, 1:H2 + 1])
    rowsR = (rV[:, :, 0][:, 0:H2], rV[:, :, 1][:, 0:H2],
             rV[:, :, 0][:, 1:H2 + 1])
    dww2 = d2wp[...]
    acc2 = jnp.zeros((2, H2, WH, C), f32)
    for dy in range(3):
        rP, rR = rowsP[dy], rowsR[dy]
        acc2 += (rR[:, :, 0:WH, :] * dww2[3 * dy + 0]
                 + rP[:, :, 0:WH, :] * dww2[3 * dy + 1]
                 + rR[:, :, 1:WH + 1, :] * dww2[3 * dy + 2])
    acc2 = jnp.clip(acc2 * d2sp[...] + d2bp[...], 0.0, 6.0)
    d2b = acc2.astype(bf16).reshape(2 * M2, C)

    b2 = jnp.dot(d2b, p2z[...], preferred_element_type=f32)
    b2v = (b2 * p2s[...] + p2b[...]).astype(bf16)

    h3 = jnp.dot(b2v, e3w[...], preferred_element_type=f32)
    h3 = jnp.clip(h3 * e3s[...] + e3b[...], 0.0, 6.0)
    hp3[:, 1:H2 + 1, 1:H2 + 1, :] = h3.reshape(2, H2, H2, C3)

    dww3 = d3w[...]
    acc3 = jnp.zeros((2, H2, H2, C3), f32)
    for dy in range(3):
        row = hp3[:, dy:dy + H2, :, :]
        for dx in range(3):
            acc3 += row[:, :, dx:dx + H2, :] * dww3[3 * dy + dx]
    acc3 = jnp.clip(acc3 * d3s[...] + d3b[...], 0.0, 6.0)
    d3v = acc3.astype(bf16).reshape(2 * M2, C3)

    b3 = jnp.dot(d3v, p3w[...], preferred_element_type=f32)
    b3 = b3 * p3s[...] + p3b[...] + b2v.astype(f32)

    hact = jnp.dot(b3.astype(bf16), hw[...], preferred_element_type=f32)
    hact = jnp.clip(hact * hs[...] + hb[...], 0.0, 6.0)
    o_ref[0:1, :, :] = (jnp.sum(hact[0:M2], axis=0, keepdims=True)
                        * (1.0 / M2)).astype(o_ref.dtype).reshape(1, 1, C)
    o_ref[1:2, :, :] = (jnp.sum(hact[M2:2 * M2], axis=0, keepdims=True)
                        * (1.0 / M2)).astype(o_ref.dtype).reshape(1, 1, C)


def _build_patches(images):
    n = images.shape[0]
    x = jnp.transpose(images, (0, 2, 3, 1))
    xp = jnp.pad(x, ((0, 0), (1, 1), (1, 3), (0, 0)))
    wins = []
    for dy in range(3):
        g = xp[:, dy:dy + 2 * H1:2, :, :].reshape(n, H1, 66, 6)
        wins.append(jnp.concatenate([g[:, :, 0:W1, :], g[:, :, 1:W1 + 1, :]],
                                    axis=3))
    patches = jnp.concatenate(wins, axis=3)
    return patches.reshape(n, MH, 2 * KP)


def _pack2(v):
    return jnp.concatenate([v[:, :CE], v[:, :CE]], axis=1)


def _pairdiag(w, zeros_like_shape=None):
    z = jnp.zeros(w.shape, w.dtype)
    return jnp.concatenate(
        [jnp.concatenate([w, z], axis=1), jnp.concatenate([z, w], axis=1)],
        axis=0)


def kernel(images, stem_w, stem_s, stem_b,
           b1_exp_w, b1_exp_s, b1_exp_b, b1_dw_w, b1_dw_s, b1_dw_b,
           b1_proj_w, b1_proj_s, b1_proj_b,
           b2_exp_w, b2_exp_s, b2_exp_b, b2_dw_w, b2_dw_s, b2_dw_b,
           b2_proj_w, b2_proj_s, b2_proj_b,
           b3_exp_w, b3_exp_s, b3_exp_b, b3_dw_w, b3_dw_s, b3_dw_b,
           b3_proj_w, b3_proj_s, b3_proj_b,
           head_w, head_s, head_b):
    n = images.shape[0]
    pcat = _build_patches(images)

    sw36 = jnp.pad(stem_w[:27, :16].reshape(3, 3, 3, 16),
                   ((0, 0), (0, 1), (0, 0), (0, 0))).reshape(KP, 16)
    swp = _pairdiag(sw36)
    pair = lambda v: jnp.concatenate([v[:, :16], v[:, :16]], axis=1)
    ssp, sbp = pair(stem_s), pair(stem_b)
    e1p = _pairdiag(b1_exp_w[:16, :CE])
    e1sp, e1bp = _pack2(b1_exp_s), _pack2(b1_exp_b)
    d1wp = _pack2(b1_dw_w)
    d1sp, d1bp = _pack2(b1_dw_s), _pack2(b1_dw_b)
    p1p = _pairdiag(b1_proj_w[:CE, :16])
    p1sp, p1bp = pair(b1_proj_s), pair(b1_proj_b)
    e2p = _pairdiag(b2_exp_w[:16, :CE])
    e2sp, e2bp = _pack2(b2_exp_s), _pack2(b2_exp_b)
    d2wp = _pack2(b2_dw_w)
    d2sp, d2bp = _pack2(b2_dw_s), _pack2(b2_dw_b)
    p2z = jnp.concatenate(
        [b2_proj_w[:CE, :24], jnp.zeros((CE, 24), jnp.bfloat16)], axis=0)
    p2s, p2b = b2_proj_s[:, :24], b2_proj_b[:, :24]
    e3w = b3_exp_w[:24, :C3]
    e3s, e3b = b3_exp_s[:, :C3], b3_exp_b[:, :C3]
    d3w = b3_dw_w[:, :C3]
    d3s, d3b = b3_dw_s[:, :C3], b3_dw_b[:, :C3]
    p3w = b3_proj_w[:C3, :24]
    p3s, p3b = b3_proj_s[:, :24], b3_proj_b[:, :24]
    hw = head_w[:24, :]

    full = lambda i: (0, 0)
    ws = lambda r, c: pl.BlockSpec((r, c), full)

    out = pl.pallas_call(
        _body,
        grid=(n // 2,),
        in_specs=[pl.BlockSpec((2, MH, 2 * KP), lambda i: (i, 0, 0)),
                  ws(2 * KP, 32), ws(1, 32), ws(1, 32),
                  ws(32, C), ws(1, C), ws(1, C),
                  ws(9, C), ws(1, C), ws(1, C),
                  ws(C, 32), ws(1, 32), ws(1, 32),
                  ws(32, C), ws(1, C), ws(1, C),
                  ws(9, C), ws(1, C), ws(1, C),
                  ws(C, 24), ws(1, 24), ws(1, 24),
                  ws(24, C3), ws(1, C3), ws(1, C3),
                  ws(9, C3), ws(1, C3), ws(1, C3),
                  ws(C3, 24), ws(1, 24), ws(1, 24),
                  ws(24, C), ws(1, C), ws(1, C)],
        out_specs=pl.BlockSpec((2, 1, C), lambda i: (i, 0, 0)),
        out_shape=jax.ShapeDtypeStruct((n, 1, C), jnp.bfloat16),
        scratch_shapes=[
            pltpu.VMEM((2, H1 + 2, WH, C), jnp.float32),
            pltpu.VMEM((2, H1 + 2, WH + 2, C), jnp.float32),
            pltpu.VMEM((2, H2 + 2, H2 + 2, C3), jnp.float32),
        ],
        compiler_params=pltpu.CompilerParams(
            dimension_semantics=("parallel",)),
    )(pcat, swp, ssp, sbp,
      e1p, e1sp, e1bp, d1wp, d1sp, d1bp, p1p, p1sp, p1bp,
      e2p, e2sp, e2bp, d2wp, d2sp, d2bp, p2z, p2s, p2b,
      e3w, e3s, e3b, d3w, d3s, d3b, p3w, p3s, p3b,
      hw, head_s, head_b)
    return out.astype(images.dtype)


# speedup vs baseline: 6.7742x; 1.0513x over previous
import jax
import jax.numpy as jnp
from jax.experimental import pallas as pl
from jax.experimental.pallas import tpu as pltpu


C = 128
H1 = 64
W1 = 64
WH = W1 // 2
M1 = H1 * W1
MH = M1 // 2
H2 = H1 // 2
M2 = H2 * WH
KP = 36
CE = 64
C3 = 96


def _body(p_ref, swp, ssp, sbp,
          e1p, e1sp, e1bp, d1wp, d1sp, d1bp, p1p, p1sp, p1bp,
          e2p, e2sp, e2bp, d2wp, d2sp, d2bp, p2z, p2s, p2b,
          e3w, e3s, e3b, d3w, d3s, d3b, p3w, p3s, p3b,
          hw, hs, hb,
          o_ref,
          hP, hR, hp3):
    f32 = jnp.float32
    bf16 = jnp.bfloat16
    mask = jax.lax.broadcasted_iota(jnp.int32, (2, H1, WH, C), 3) < CE

    hP[:, 0:1, :, :] = jnp.zeros((2, 1, WH, C), f32)
    hP[:, H1 + 1:H1 + 2, :, :] = jnp.zeros((2, 1, WH, C), f32)
    hR[:, 0:1, :, :] = jnp.zeros((2, 1, WH + 2, C), f32)
    hR[:, H1 + 1:H1 + 2, :, :] = jnp.zeros((2, 1, WH + 2, C), f32)
    hR[:, :, 0:1, :] = jnp.zeros((2, H1 + 2, 1, C), f32)
    hR[:, :, WH + 1:WH + 2, :] = jnp.zeros((2, H1 + 2, 1, C), f32)
    hp3[:, 0:1, :, :] = jnp.zeros((2, 1, H2 + 2, C3), f32)
    hp3[:, H2 + 1:H2 + 2, :, :] = jnp.zeros((2, 1, H2 + 2, C3), f32)
    hp3[:, :, 0:1, :] = jnp.zeros((2, H2 + 2, 1, C3), f32)
    hp3[:, :, H2 + 1:H2 + 2, :] = jnp.zeros((2, H2 + 2, 1, C3), f32)

    s = jnp.dot(p_ref[...].reshape(2 * MH, 2 * KP), swp[...],
                preferred_element_type=f32)
    spb = jnp.clip(s * ssp[...] + sbp[...], 0.0, 6.0).astype(bf16)

    def packed_dw_input(P, scale, bias):
        P = jnp.clip(P * scale + bias, 0.0, 6.0)
        R = jnp.concatenate([P[:, CE:C], P[:, 0:CE]], axis=1)
        hP[:, 1:H1 + 1, 0:WH, :] = P.reshape(2, H1, WH, C)
        hR[:, 1:H1 + 1, 1:WH + 1, :] = R.reshape(2, H1, WH, C)

    packed_dw_input(jnp.dot(spb, e1p[...], preferred_element_type=f32),
                    e1sp[...], e1bp[...])
    dww = d1wp[...]
    acc = jnp.zeros((2, H1, WH, C), f32)
    for dy in range(3):
        rP = hP[:, dy:dy + H1, :, :]
        rR = hR[:, dy:dy + H1, :, :]
        A0 = jnp.where(mask, rR[:, :, 0:WH, :], rR[:, :, 1:WH + 1, :])
        A2 = jnp.where(mask, rR[:, :, 1:WH + 1, :], rR[:, :, 2:WH + 2, :])
        acc += (A0 * dww[3 * dy + 0] + rP * dww[3 * dy + 1]
                + A2 * dww[3 * dy + 2])
    acc = jnp.clip(acc * d1sp[...] + d1bp[...], 0.0, 6.0)
    OPb = acc.astype(bf16).reshape(2 * MH, C)

    y = jnp.dot(OPb, p1p[...], preferred_element_type=f32)
    yp = ((y * p1sp[...] + p1bp[...]) + spb.astype(f32)).astype(bf16)

    packed_dw_input(jnp.dot(yp, e2p[...], preferred_element_type=f32),
                    e2sp[...], e2bp[...])
    pV = hP[...].reshape(2, (H1 + 2) // 2, 2, WH, C)
    rV = hR[...].reshape(2, (H1 + 2) // 2, 2, WH + 2, C)
    rowsP = (pV[:, :, 0][:, 0:H2], pV[:, :, 1][:, 0:H2],
             pV[:, :, 0][:, 1:H2 + 1])
    rowsR = (rV[:, :, 0][:, 0:H2], rV[:, :, 1][:, 0:H2],
             rV[:, :, 0][:, 1:H2 + 1])
    dww2 = d2wp[...]
    acc2 = jnp.zeros((2, H2, WH, C), f32)
    for dy in range(3):
        rP, rR = rowsP[dy], rowsR[dy]
        acc2 += (rR[:, :, 0:WH, :] * dww2[3 * dy + 0]
                 + rP[:, :, 0:WH, :] * dww2[3 * dy + 1]
                 + rR[:, :, 1:WH + 1, :] * dww2[3 * dy + 2])
    acc2 = jnp.clip(acc2 * d2sp[...] + d2bp[...], 0.0, 6.0)
    d2b = acc2.astype(bf16).reshape(2 * M2, C)

    b2 = jnp.dot(d2b, p2z[...], preferred_element_type=f32)
    b2v = (b2 * p2s[...] + p2b[...]).astype(bf16)

    h3 = jnp.dot(b2v, e3w[...], preferred_element_type=f32)
    h3 = jnp.clip(h3 * e3s[...] + e3b[...], 0.0, 6.0)
    hp3[:, 1:H2 + 1, 1:H2 + 1, :] = h3.reshape(2, H2, H2, C3)

    dww3 = d3w[...]
    acc3 = jnp.zeros((2, H2, H2, C3), f32)
    for dy in range(3):
        row = hp3[:, dy:dy + H2, :, :]
        for dx in range(3):
            acc3 += row[:, :, dx:dx + H2, :] * dww3[3 * dy + dx]
    acc3 = jnp.clip(acc3 * d3s[...] + d3b[...], 0.0, 6.0)
    d3v = acc3.astype(bf16).reshape(2 * M2, C3)

    b3 = jnp.dot(d3v, p3w[...], preferred_element_type=f32)
    b3 = b3 * p3s[...] + p3b[...] + b2v.astype(f32)

    hact = jnp.dot(b3.astype(bf16), hw[...], preferred_element_type=f32)
    hact = jnp.clip(hact * hs[...] + hb[...], 0.0, 6.0)
    o_ref[0:1, :, :] = (jnp.sum(hact[0:M2], axis=0, keepdims=True)
                        * (1.0 / M2)).astype(o_ref.dtype).reshape(1, 1, C)
    o_ref[1:2, :, :] = (jnp.sum(hact[M2:2 * M2], axis=0, keepdims=True)
                        * (1.0 / M2)).astype(o_ref.dtype).reshape(1, 1, C)


def _build_patches(images):
    n = images.shape[0]
    xp = jnp.pad(images, ((0, 0), (0, 0), (1, 1), (1, 3)))
    wins = []
    for dy in range(3):
        for c in range(3):
            g = xp[:, c, dy:dy + 2 * H1:2, :].reshape(n, H1, 66, 2)
            wins.append(jnp.concatenate(
                [g[:, :, 0:W1, :], g[:, :, 1:W1 + 1, :]],
                axis=3))
    patches = jnp.concatenate(wins, axis=3)
    return patches.reshape(n, MH, 2 * KP)


def _pack2(v):
    return jnp.concatenate([v[:, :CE], v[:, :CE]], axis=1)


def _pairdiag(w, zeros_like_shape=None):
    z = jnp.zeros(w.shape, w.dtype)
    return jnp.concatenate(
        [jnp.concatenate([w, z], axis=1), jnp.concatenate([z, w], axis=1)],
        axis=0)


def kernel(images, stem_w, stem_s, stem_b,
           b1_exp_w, b1_exp_s, b1_exp_b, b1_dw_w, b1_dw_s, b1_dw_b,
           b1_proj_w, b1_proj_s, b1_proj_b,
           b2_exp_w, b2_exp_s, b2_exp_b, b2_dw_w, b2_dw_s, b2_dw_b,
           b2_proj_w, b2_proj_s, b2_proj_b,
           b3_exp_w, b3_exp_s, b3_exp_b, b3_dw_w, b3_dw_s, b3_dw_b,
           b3_proj_w, b3_proj_s, b3_proj_b,
           head_w, head_s, head_b):
    n = images.shape[0]
    pcat = _build_patches(images)

    sw36 = jnp.pad(
        jnp.transpose(stem_w[:27, :16].reshape(3, 3, 3, 16), (0, 2, 1, 3)),
        ((0, 0), (0, 0), (0, 1), (0, 0))).reshape(KP, 16)
    swp = _pairdiag(sw36)
    pair = lambda v: jnp.concatenate([v[:, :16], v[:, :16]], axis=1)
    ssp, sbp = pair(stem_s), pair(stem_b)
    e1p = _pairdiag(b1_exp_w[:16, :CE])
    e1sp, e1bp = _pack2(b1_exp_s), _pack2(b1_exp_b)
    d1wp = _pack2(b1_dw_w)
    d1sp, d1bp = _pack2(b1_dw_s), _pack2(b1_dw_b)
    p1p = _pairdiag(b1_proj_w[:CE, :16])
    p1sp, p1bp = pair(b1_proj_s), pair(b1_proj_b)
    e2p = _pairdiag(b2_exp_w[:16, :CE])
    e2sp, e2bp = _pack2(b2_exp_s), _pack2(b2_exp_b)
    d2wp = _pack2(b2_dw_w)
    d2sp, d2bp = _pack2(b2_dw_s), _pack2(b2_dw_b)
    p2z = jnp.concatenate(
        [b2_proj_w[:CE, :24], jnp.zeros((CE, 24), jnp.bfloat16)], axis=0)
    p2s, p2b = b2_proj_s[:, :24], b2_proj_b[:, :24]
    e3w = b3_exp_w[:24, :C3]
    e3s, e3b = b3_exp_s[:, :C3], b3_exp_b[:, :C3]
    d3w = b3_dw_w[:, :C3]
    d3s, d3b = b3_dw_s[:, :C3], b3_dw_b[:, :C3]
    p3w = b3_proj_w[:C3, :24]
    p3s, p3b = b3_proj_s[:, :24], b3_proj_b[:, :24]
    hw = head_w[:24, :]

    full = lambda i: (0, 0)
    ws = lambda r, c: pl.BlockSpec((r, c), full)

    out = pl.pallas_call(
        _body,
        grid=(n // 2,),
        in_specs=[pl.BlockSpec((2, MH, 2 * KP), lambda i: (i, 0, 0)),
                  ws(2 * KP, 32), ws(1, 32), ws(1, 32),
                  ws(32, C), ws(1, C), ws(1, C),
                  ws(9, C), ws(1, C), ws(1, C),
                  ws(C, 32), ws(1, 32), ws(1, 32),
                  ws(32, C), ws(1, C), ws(1, C),
                  ws(9, C), ws(1, C), ws(1, C),
                  ws(C, 24), ws(1, 24), ws(1, 24),
                  ws(24, C3), ws(1, C3), ws(1, C3),
                  ws(9, C3), ws(1, C3), ws(1, C3),
                  ws(C3, 24), ws(1, 24), ws(1, 24),
                  ws(24, C), ws(1, C), ws(1, C)],
        out_specs=pl.BlockSpec((2, 1, C), lambda i: (i, 0, 0)),
        out_shape=jax.ShapeDtypeStruct((n, 1, C), jnp.bfloat16),
        scratch_shapes=[
            pltpu.VMEM((2, H1 + 2, WH, C), jnp.float32),
            pltpu.VMEM((2, H1 + 2, WH + 2, C), jnp.float32),
            pltpu.VMEM((2, H2 + 2, H2 + 2, C3), jnp.float32),
        ],
        compiler_params=pltpu.CompilerParams(
            dimension_semantics=("parallel",)),
    )(pcat, swp, ssp, sbp,
      e1p, e1sp, e1bp, d1wp, d1sp, d1bp, p1p, p1sp, p1bp,
      e2p, e2sp, e2bp, d2wp, d2sp, d2bp, p2z, p2s, p2b,
      e3w, e3s, e3b, d3w, d3s, d3b, p3w, p3s, p3b,
      hw, head_s, head_b)
    return out.astype(images.dtype)
